# baseline (speedup 1.0000x reference)
import functools

import jax
import jax.numpy as jnp
from jax import lax
from jax.experimental import pallas as pl
from jax.experimental.pallas import tpu as pltpu
from jax.experimental.pallas import tpu_sc as plsc

N = 10000
E = 320000
D = 128
HD = D // 2
ED = 10

NC = 2
NS = 16
LANES = 16
NW = NC * NS
EPW = 10240
E_PAD = NW * EPW
PAD_DST = 10200
CHUNK = 128
NCHUNK = EPW // CHUNK
NPAD = 10240
NACC = NPAD // 2
ROWS_PER_TILE = NACC // NS
STAGE_ROWS = 64


def _tables_body(f_ref, w0_ref, w1_ref, w2_ref, w3_ref,
                 t0_ref, t1_ref, u0_ref, u1_ref):
    x = f_ref[...]
    t0_ref[...] = jnp.dot(x, w0_ref[...], preferred_element_type=jnp.float32)
    t1_ref[...] = jnp.dot(x, w1_ref[...], preferred_element_type=jnp.float32)
    u0_ref[...] = jnp.dot(x, w2_ref[...], preferred_element_type=jnp.float32)
    u1_ref[...] = jnp.dot(x, w3_ref[...], preferred_element_type=jnp.float32)


def _node_tables(feature, w0, w1, w2, w3):
    rb = 1000
    grid = N // rb
    wspec = pl.BlockSpec((D, D), lambda i: (0, 0))
    ospec = pl.BlockSpec((rb, D), lambda i: (i, 0))
    oshape = jax.ShapeDtypeStruct((N, D), jnp.float32)
    return pl.pallas_call(
        _tables_body,
        grid=(grid,),
        in_specs=[pl.BlockSpec((rb, D), lambda i: (i, 0)),
                  wspec, wspec, wspec, wspec],
        out_specs=[ospec, ospec, ospec, ospec],
        out_shape=[oshape, oshape, oshape, oshape],
    )(feature, w0, w1, w2, w3)


def _edge_proj_body(ed_ref, wd0_ref, wd1_ref, b0_ref, b1_ref,
                    dd0_ref, dd1_ref):
    x = ed_ref[...]
    dd0_ref[...] = (
        jnp.dot(x, wd0_ref[...], preferred_element_type=jnp.float32)
        + b0_ref[...])
    dd1_ref[...] = (
        jnp.dot(x, wd1_ref[...], preferred_element_type=jnp.float32)
        + b1_ref[...])


def _edge_proj(edge_dist, wd0, wd1, b0, b1):
    eb = 4096
    grid = E_PAD // eb
    wspec = pl.BlockSpec((ED, D), lambda i: (0, 0))
    bspec = pl.BlockSpec((1, D), lambda i: (0, 0))
    ospec = pl.BlockSpec((eb, D), lambda i: (i, 0))
    oshape = jax.ShapeDtypeStruct((E_PAD, D), jnp.float32)
    return pl.pallas_call(
        _edge_proj_body,
        grid=(grid,),
        in_specs=[pl.BlockSpec((eb, ED), lambda i: (i, 0)),
                  wspec, wspec, bspec, bspec],
        out_specs=[ospec, ospec],
        out_shape=[oshape, oshape],
    )(edge_dist, wd0, wd1, b0, b1)


def _sigmoid(x):
    return 1.0 / (1.0 + jnp.exp(-x))


def _softplus(x):
    z = jnp.exp(-jnp.abs(x))
    w = z / (2.0 + z)
    u = w * w
    poly = 2.0 + u * (2.0 / 3.0 + u * (2.0 / 5.0 + u * (2.0 / 7.0)))
    return jnp.maximum(x, 0.0) + w * poly


def _sc_body(t0_hbm, t1_hbm, u0_hbm, u1_hbm, dd0_hbm, dd1_hbm,
             src_hbm, dst_hbm, d2_hbm, hm_hbm, zeros_hbm, out_hbm,
             sidx, didx, d2idx, hmv, trows, urows, ddv, mv, stage, acc,
             sem_a, sem_b):
    c = lax.axis_index("c")
    s = lax.axis_index("s")
    w = c * NS + s
    my_rows = s * ROWS_PER_TILE
    wbase = w * EPW

    for p, (t_hbm, u_hbm, dd_hbm) in enumerate(
            ((t0_hbm, u0_hbm, dd0_hbm), (t1_hbm, u1_hbm, dd1_hbm))):
        pltpu.sync_copy(zeros_hbm, stage)
        for k in range(ROWS_PER_TILE // STAGE_ROWS):
            pltpu.sync_copy(
                stage, acc.at[pl.ds(my_rows + k * STAGE_ROWS, STAGE_ROWS)])
        plsc.subcore_barrier()

        @pl.loop(0, NCHUNK)
        def chunk_body(i):
            base = pl.multiple_of(wbase + i * CHUNK, CHUNK)
            pltpu.sync_copy(src_hbm.at[pl.ds(base, CHUNK)], sidx)
            pltpu.sync_copy(dst_hbm.at[pl.ds(base, CHUNK)], didx)
            pltpu.sync_copy(d2_hbm.at[pl.ds(base, CHUNK)], d2idx)
            ga = pltpu.async_copy(t_hbm.at[sidx], trows, sem_a)
            gb = pltpu.async_copy(u_hbm.at[didx], urows, sem_b)
            pltpu.sync_copy(dd_hbm.at[pl.ds(base, CHUNK)], ddv)
            hbase = pl.multiple_of(base // 8, CHUNK // 8)
            pltpu.sync_copy(hm_hbm.at[pl.ds(hbase, CHUNK // 8)], hmv)
            ga.wait()
            gb.wait()

            def edge_body(e, carry2):
                hf = hmv[e >> 3, pl.ds((e & 7) * LANES, LANES)]
                cf = 1.0 - hf
                for v in range(HD // LANES):
                    lo = v * LANES
                    g = (trows[e, pl.ds(lo, LANES)]
                         + urows[e, pl.ds(lo, LANES)]
                         + ddv[e, pl.ds(lo, LANES)])
                    x = (trows[e, pl.ds(HD + lo, LANES)]
                         + urows[e, pl.ds(HD + lo, LANES)]
                         + ddv[e, pl.ds(HD + lo, LANES)])
                    m = _sigmoid(g) * _softplus(x)
                    mv[e, pl.ds(lo, LANES)] = m * cf
                    mv[e, pl.ds(HD + lo, LANES)] = m * hf
                return carry2

            lax.fori_loop(0, CHUNK, edge_body, 0)
            pltpu.sync_copy(mv, acc.at[d2idx], add=True)

        plsc.subcore_barrier()

        for k in range(ROWS_PER_TILE // STAGE_ROWS):
            rs = my_rows + k * STAGE_ROWS
            pltpu.sync_copy(acc.at[pl.ds(rs, STAGE_ROWS)], stage)
            obase = pl.multiple_of((c * 2 + p) * NACC + rs, STAGE_ROWS)
            pltpu.sync_copy(stage, out_hbm.at[pl.ds(obase, STAGE_ROWS)])
        plsc.subcore_barrier()


_sc_edges = functools.partial(
    pl.kernel,
    out_type=jax.ShapeDtypeStruct((NC * 2 * NACC, D), jnp.float32),
    mesh=plsc.VectorSubcoreMesh(core_axis_name="c", subcore_axis_name="s"),
    scratch_types=[
        pltpu.VMEM((CHUNK,), jnp.int32),
        pltpu.VMEM((CHUNK,), jnp.int32),
        pltpu.VMEM((CHUNK,), jnp.int32),
        pltpu.VMEM((CHUNK // 8, D), jnp.float32),
        pltpu.VMEM((CHUNK, D), jnp.float32),
        pltpu.VMEM((CHUNK, D), jnp.float32),
        pltpu.VMEM((CHUNK, D), jnp.float32),
        pltpu.VMEM((CHUNK, D), jnp.float32),
        pltpu.VMEM((STAGE_ROWS, D), jnp.float32),
        pltpu.VMEM_SHARED((NACC, D), jnp.float32),
        pltpu.SemaphoreType.DMA,
        pltpu.SemaphoreType.DMA,
    ],
)(_sc_body)


def _final_body(f_ref, p_ref, o_ref):
    lo = p_ref[0, 0] + p_ref[1, 0]
    hi = p_ref[0, 1] + p_ref[1, 1]
    o_ref[...] = f_ref[...] + jnp.concatenate([lo, hi], axis=-1)


def _final_add(feature, partials):
    rb = 1000
    grid = N // rb
    partials = partials.reshape(NC, 2, NPAD, HD)
    return pl.pallas_call(
        _final_body,
        grid=(grid,),
        in_specs=[
            pl.BlockSpec((rb, D), lambda i: (i, 0)),
            pl.BlockSpec((NC, 2, rb, HD), lambda i: (0, 0, i, 0)),
        ],
        out_specs=pl.BlockSpec((rb, D), lambda i: (i, 0)),
        out_shape=jax.ShapeDtypeStruct((N, D), jnp.float32),
    )(feature, partials)


def kernel(feature, edge_index, edge_dist, Wf, bf, Ws, bs):
    f32 = jnp.float32
    feature = feature.astype(f32)
    npad = E_PAD - E
    src = jnp.concatenate(
        [edge_index[0].astype(jnp.int32), jnp.zeros((npad,), jnp.int32)])
    dst = jnp.concatenate(
        [edge_index[1].astype(jnp.int32),
         jnp.full((npad,), PAD_DST, jnp.int32)])
    d2 = lax.shift_right_logical(dst, 1)
    hmask = jnp.repeat((dst & 1).astype(f32), LANES).reshape(E_PAD // 8, D)
    edge_dist_p = jnp.concatenate(
        [edge_dist.astype(f32), jnp.zeros((npad, ED), f32)])

    def wpair(lo):
        return jnp.concatenate([Wf[lo:lo + HD], Ws[lo:lo + HD]],
                               axis=0).astype(f32)

    w_t0 = wpair(0)[:, :D].T
    w_t1 = wpair(HD)[:, :D].T
    w_u0 = wpair(0)[:, D:2 * D].T
    w_u1 = wpair(HD)[:, D:2 * D].T
    wd0 = wpair(0)[:, 2 * D:].T
    wd1 = wpair(HD)[:, 2 * D:].T
    b0 = jnp.concatenate([bf[:HD], bs[:HD]]).reshape(1, D).astype(f32)
    b1 = jnp.concatenate([bf[HD:], bs[HD:]]).reshape(1, D).astype(f32)

    t0, t1, u0, u1 = _node_tables(feature, w_t0, w_t1, w_u0, w_u1)
    dd0, dd1 = _edge_proj(edge_dist_p, wd0, wd1, b0, b1)
    zeros = jnp.zeros((STAGE_ROWS, D), f32)
    partials = _sc_edges(t0, t1, u0, u1, dd0, dd1, src, dst, d2, hmask, zeros)
    return _final_add(feature, partials)


# speedup vs baseline: 1.1367x; 1.1367x over previous
import functools

import jax
import jax.numpy as jnp
from jax import lax
from jax.experimental import pallas as pl
from jax.experimental.pallas import tpu as pltpu
from jax.experimental.pallas import tpu_sc as plsc

N = 10000
E = 320000
D = 128
HD = D // 2
ED = 10

NC = 2
NS = 16
LANES = 16
NW = NC * NS
EPW = 10240
E_PAD = NW * EPW
PAD_DST = 10200
CHUNK = 80
NCHUNK = EPW // CHUNK
NPAD = 10240
NACC = NPAD // 2
ROWS_PER_TILE = NACC // NS
STAGE_ROWS = 32


def _tables_body(f_ref, w0_ref, w1_ref, w2_ref, w3_ref,
                 t0_ref, t1_ref, u0_ref, u1_ref):
    x = f_ref[...]
    t0_ref[...] = jnp.dot(x, w0_ref[...], preferred_element_type=jnp.float32)
    t1_ref[...] = jnp.dot(x, w1_ref[...], preferred_element_type=jnp.float32)
    u0_ref[...] = jnp.dot(x, w2_ref[...], preferred_element_type=jnp.float32)
    u1_ref[...] = jnp.dot(x, w3_ref[...], preferred_element_type=jnp.float32)


def _node_tables(feature, w0, w1, w2, w3):
    rb = 1000
    grid = N // rb
    wspec = pl.BlockSpec((D, D), lambda i: (0, 0))
    ospec = pl.BlockSpec((rb, D), lambda i: (i, 0))
    oshape = jax.ShapeDtypeStruct((N, D), jnp.float32)
    return pl.pallas_call(
        _tables_body,
        grid=(grid,),
        in_specs=[pl.BlockSpec((rb, D), lambda i: (i, 0)),
                  wspec, wspec, wspec, wspec],
        out_specs=[ospec, ospec, ospec, ospec],
        out_shape=[oshape, oshape, oshape, oshape],
    )(feature, w0, w1, w2, w3)


def _edge_proj_body(ed_ref, wd0_ref, wd1_ref, b0_ref, b1_ref,
                    dd0_ref, dd1_ref):
    x = ed_ref[...]
    dd0_ref[...] = (
        jnp.dot(x, wd0_ref[...], preferred_element_type=jnp.float32)
        + b0_ref[...])
    dd1_ref[...] = (
        jnp.dot(x, wd1_ref[...], preferred_element_type=jnp.float32)
        + b1_ref[...])


def _edge_proj(edge_dist, wd0, wd1, b0, b1):
    eb = 4096
    grid = E_PAD // eb
    wspec = pl.BlockSpec((ED, D), lambda i: (0, 0))
    bspec = pl.BlockSpec((1, D), lambda i: (0, 0))
    ospec = pl.BlockSpec((eb, D), lambda i: (i, 0))
    oshape = jax.ShapeDtypeStruct((E_PAD, D), jnp.float32)
    return pl.pallas_call(
        _edge_proj_body,
        grid=(grid,),
        in_specs=[pl.BlockSpec((eb, ED), lambda i: (i, 0)),
                  wspec, wspec, bspec, bspec],
        out_specs=[ospec, ospec],
        out_shape=[oshape, oshape],
    )(edge_dist, wd0, wd1, b0, b1)


def _sigmoid(x):
    return 1.0 / (1.0 + jnp.exp(-x))


def _softplus(x):
    z = jnp.exp(-jnp.abs(x))
    w = z / (2.0 + z)
    u = w * w
    poly = 2.0 + u * (2.0 / 3.0 + u * (2.0 / 5.0 + u * (2.0 / 7.0)))
    return jnp.maximum(x, 0.0) + w * poly


def _sc_body(t0_hbm, t1_hbm, u0_hbm, u1_hbm, dd0_hbm, dd1_hbm,
             src_hbm, dst_hbm, d2_hbm, hm_hbm, zeros_hbm, out_hbm,
             sidx, didx, d2idx, hmv, trows, urows, ddv, mv, stage, acc,
             sem_t, sem_u, sem_i, sem_l):
    c = lax.axis_index("c")
    s = lax.axis_index("s")
    w = c * NS + s
    my_rows = s * ROWS_PER_TILE
    wbase = w * EPW

    for p, (t_hbm, u_hbm, dd_hbm) in enumerate(
            ((t0_hbm, u0_hbm, dd0_hbm), (t1_hbm, u1_hbm, dd1_hbm))):
        pltpu.sync_copy(zeros_hbm, stage)
        for k in range(ROWS_PER_TILE // STAGE_ROWS):
            pltpu.sync_copy(
                stage, acc.at[pl.ds(my_rows + k * STAGE_ROWS, STAGE_ROWS)])
        plsc.subcore_barrier()

        def cbase(g):
            return pl.multiple_of(wbase + g * CHUNK, CHUNK)

        def issue_idx(g, b):
            base = cbase(g)
            pltpu.async_copy(src_hbm.at[pl.ds(base, CHUNK)], sidx[b],
                             sem_i.at[b])
            pltpu.async_copy(dst_hbm.at[pl.ds(base, CHUNK)], didx[b],
                             sem_i.at[b])
            pltpu.async_copy(d2_hbm.at[pl.ds(base, CHUNK)], d2idx[b],
                             sem_i.at[b])

        def wait_idx(g, b):
            base = cbase(g)
            pltpu.make_async_copy(src_hbm.at[pl.ds(base, CHUNK)], sidx[b],
                                  sem_i.at[b]).wait()
            pltpu.make_async_copy(dst_hbm.at[pl.ds(base, CHUNK)], didx[b],
                                  sem_i.at[b]).wait()
            pltpu.make_async_copy(d2_hbm.at[pl.ds(base, CHUNK)], d2idx[b],
                                  sem_i.at[b]).wait()

        def issue_main(g, b):
            base = cbase(g)
            hbase = pl.multiple_of(base * LANES, CHUNK * LANES)
            pltpu.async_copy(t_hbm.at[sidx[b]], trows[b], sem_t.at[b])
            pltpu.async_copy(u_hbm.at[didx[b]], urows[b], sem_u.at[b])
            pltpu.async_copy(dd_hbm.at[pl.ds(base, CHUNK)], ddv[b],
                             sem_l.at[b])
            pltpu.async_copy(hm_hbm.at[pl.ds(hbase, CHUNK * LANES)], hmv[b],
                             sem_l.at[b])

        def wait_main(g, b):
            base = cbase(g)
            hbase = pl.multiple_of(base * LANES, CHUNK * LANES)
            pltpu.make_async_copy(t_hbm.at[sidx[b]], trows[b],
                                  sem_t.at[b]).wait()
            pltpu.make_async_copy(u_hbm.at[didx[b]], urows[b],
                                  sem_u.at[b]).wait()
            pltpu.make_async_copy(dd_hbm.at[pl.ds(base, CHUNK)], ddv[b],
                                  sem_l.at[b]).wait()
            pltpu.make_async_copy(hm_hbm.at[pl.ds(hbase, CHUNK * LANES)],
                                  hmv[b], sem_l.at[b]).wait()

        issue_idx(0, 0)
        issue_idx(1, 1)
        wait_idx(0, 0)
        issue_main(0, 0)

        @pl.loop(0, NCHUNK, step=2)
        def chunk_body(i):
            for b in range(2):
                g = i + b
                wait_main(g, b)

                @pl.when(g + 1 < NCHUNK)
                def _():
                    wait_idx(g + 1, 1 - b)
                    issue_main(g + 1, 1 - b)

                def edge_body(e, carry2):
                    hf = hmv[b][pl.ds(e * LANES, LANES)]
                    cf = 1.0 - hf
                    for v in range(HD // LANES):
                        lo = v * LANES
                        g_ = (trows[b][e, pl.ds(lo, LANES)]
                              + urows[b][e, pl.ds(lo, LANES)]
                              + ddv[b][e, pl.ds(lo, LANES)])
                        x = (trows[b][e, pl.ds(HD + lo, LANES)]
                             + urows[b][e, pl.ds(HD + lo, LANES)]
                             + ddv[b][e, pl.ds(HD + lo, LANES)])
                        m = _sigmoid(g_) * _softplus(x)
                        mv[e, pl.ds(lo, LANES)] = m * cf
                        mv[e, pl.ds(HD + lo, LANES)] = m * hf
                    return carry2

                lax.fori_loop(0, CHUNK, edge_body, 0, unroll=2)
                pltpu.sync_copy(mv, acc.at[d2idx[b]], add=True)

                @pl.when(g + 2 < NCHUNK)
                def _():
                    issue_idx(g + 2, b)

        plsc.subcore_barrier()

        for k in range(ROWS_PER_TILE // STAGE_ROWS):
            rs = my_rows + k * STAGE_ROWS
            pltpu.sync_copy(acc.at[pl.ds(rs, STAGE_ROWS)], stage)
            obase = pl.multiple_of((c * 2 + p) * NACC + rs, STAGE_ROWS)
            pltpu.sync_copy(stage, out_hbm.at[pl.ds(obase, STAGE_ROWS)])
        plsc.subcore_barrier()


_sc_edges = functools.partial(
    pl.kernel,
    out_type=jax.ShapeDtypeStruct((NC * 2 * NACC, D), jnp.float32),
    mesh=plsc.VectorSubcoreMesh(core_axis_name="c", subcore_axis_name="s"),
    scratch_types=[
        [pltpu.VMEM((CHUNK,), jnp.int32)] * 2,
        [pltpu.VMEM((CHUNK,), jnp.int32)] * 2,
        [pltpu.VMEM((CHUNK,), jnp.int32)] * 2,
        [pltpu.VMEM((CHUNK * LANES,), jnp.float32)] * 2,
        [pltpu.VMEM((CHUNK, D), jnp.float32)] * 2,
        [pltpu.VMEM((CHUNK, D), jnp.float32)] * 2,
        [pltpu.VMEM((CHUNK, D), jnp.float32)] * 2,
        pltpu.VMEM((CHUNK, D), jnp.float32),
        pltpu.VMEM((STAGE_ROWS, D), jnp.float32),
        pltpu.VMEM_SHARED((NACC, D), jnp.float32),
        pltpu.SemaphoreType.DMA((2,)),
        pltpu.SemaphoreType.DMA((2,)),
        pltpu.SemaphoreType.DMA((2,)),
        pltpu.SemaphoreType.DMA((2,)),
    ],
)(_sc_body)


def _final_body(f_ref, p_ref, o_ref):
    lo = p_ref[0, 0] + p_ref[1, 0]
    hi = p_ref[0, 1] + p_ref[1, 1]
    o_ref[...] = f_ref[...] + jnp.concatenate([lo, hi], axis=-1)


def _final_add(feature, partials):
    rb = 1000
    grid = N // rb
    partials = partials.reshape(NC, 2, NPAD, HD)
    return pl.pallas_call(
        _final_body,
        grid=(grid,),
        in_specs=[
            pl.BlockSpec((rb, D), lambda i: (i, 0)),
            pl.BlockSpec((NC, 2, rb, HD), lambda i: (0, 0, i, 0)),
        ],
        out_specs=pl.BlockSpec((rb, D), lambda i: (i, 0)),
        out_shape=jax.ShapeDtypeStruct((N, D), jnp.float32),
    )(feature, partials)


def kernel(feature, edge_index, edge_dist, Wf, bf, Ws, bs):
    f32 = jnp.float32
    feature = feature.astype(f32)
    npad = E_PAD - E
    src = jnp.concatenate(
        [edge_index[0].astype(jnp.int32), jnp.zeros((npad,), jnp.int32)])
    dst = jnp.concatenate(
        [edge_index[1].astype(jnp.int32),
         jnp.full((npad,), PAD_DST, jnp.int32)])
    d2 = lax.shift_right_logical(dst, 1)
    hmask = jnp.repeat((dst & 1).astype(f32), LANES)
    edge_dist_p = jnp.concatenate(
        [edge_dist.astype(f32), jnp.zeros((npad, ED), f32)])

    def wpair(lo):
        return jnp.concatenate([Wf[lo:lo + HD], Ws[lo:lo + HD]],
                               axis=0).astype(f32)

    w_t0 = wpair(0)[:, :D].T
    w_t1 = wpair(HD)[:, :D].T
    w_u0 = wpair(0)[:, D:2 * D].T
    w_u1 = wpair(HD)[:, D:2 * D].T
    wd0 = wpair(0)[:, 2 * D:].T
    wd1 = wpair(HD)[:, 2 * D:].T
    b0 = jnp.concatenate([bf[:HD], bs[:HD]]).reshape(1, D).astype(f32)
    b1 = jnp.concatenate([bf[HD:], bs[HD:]]).reshape(1, D).astype(f32)

    t0, t1, u0, u1 = _node_tables(feature, w_t0, w_t1, w_u0, w_u1)
    dd0, dd1 = _edge_proj(edge_dist_p, wd0, wd1, b0, b1)
    zeros = jnp.zeros((STAGE_ROWS, D), f32)
    partials = _sc_edges(t0, t1, u0, u1, dd0, dd1, src, dst, d2, hmask, zeros)
    return _final_add(feature, partials)


# speedup vs baseline: 1.8291x; 1.6091x over previous
import functools

import numpy as np

import jax
import jax.numpy as jnp
from jax import lax
from jax.experimental import pallas as pl
from jax.experimental.pallas import tpu as pltpu
from jax.experimental.pallas import tpu_sc as plsc

N = 10000
E = 320000
D = 128
ED = 10

NC = 2
NS = 16
LANES = 16
NW = NC * NS
EPW = 10240
E_PAD = NW * EPW
PAD_DST = 10200
CHUNK = 40
NCHUNK = EPW // CHUNK
NPAD = 10240
ROWS_PER_TILE = NPAD // NS
STAGE_ROWS = 16
TWO_D = 2 * D


def _pack_pair(a, b):
    au = lax.bitcast_convert_type(a.astype(jnp.bfloat16), jnp.uint16)
    bu = lax.bitcast_convert_type(b.astype(jnp.bfloat16), jnp.uint16)
    return au.astype(jnp.uint32) | (bu.astype(jnp.uint32) << 16)


def _tables_body(f_ref, wgs_ref, wcs_ref, wgd_ref, wcd_ref, t_ref, u_ref):
    x = f_ref[...]
    gs = jnp.dot(x, wgs_ref[...], preferred_element_type=jnp.float32)
    cs = jnp.dot(x, wcs_ref[...], preferred_element_type=jnp.float32)
    gd = jnp.dot(x, wgd_ref[...], preferred_element_type=jnp.float32)
    cd = jnp.dot(x, wcd_ref[...], preferred_element_type=jnp.float32)
    t_ref[...] = _pack_pair(gs, cs)
    u_ref[...] = _pack_pair(gd, cd)


def _node_tables(feature, wgs, wcs, wgd, wcd):
    rb = 1000
    grid = N // rb
    wspec = pl.BlockSpec((D, D), lambda i: (0, 0))
    ospec = pl.BlockSpec((rb, D), lambda i: (i, 0))
    oshape = jax.ShapeDtypeStruct((N, D), jnp.uint32)
    return pl.pallas_call(
        _tables_body,
        grid=(grid,),
        in_specs=[pl.BlockSpec((rb, D), lambda i: (i, 0)),
                  wspec, wspec, wspec, wspec],
        out_specs=[ospec, ospec],
        out_shape=[oshape, oshape],
    )(feature, wgs, wcs, wgd, wcd)


def _edge_proj_body(ed_ref, wg_ref, wc_ref, bg_ref, bc_ref, dd_ref):
    x = ed_ref[...]
    g = jnp.dot(x, wg_ref[...], preferred_element_type=jnp.float32) + bg_ref[...]
    c = jnp.dot(x, wc_ref[...], preferred_element_type=jnp.float32) + bc_ref[...]
    dd_ref[...] = _pack_pair(g, c)


def _edge_proj(edge_dist, wg, wc, bg, bc):
    eb = 4096
    grid = E_PAD // eb
    wspec = pl.BlockSpec((ED, D), lambda i: (0, 0))
    bspec = pl.BlockSpec((1, D), lambda i: (0, 0))
    return pl.pallas_call(
        _edge_proj_body,
        grid=(grid,),
        in_specs=[pl.BlockSpec((eb, ED), lambda i: (i, 0)),
                  wspec, wspec, bspec, bspec],
        out_specs=pl.BlockSpec((eb, D), lambda i: (i, 0)),
        out_shape=jax.ShapeDtypeStruct((E_PAD, D), jnp.uint32),
    )(edge_dist, wg, wc, bg, bc)


_HI_MASK = np.uint32(0xFFFF0000)


def _unpack_lo(v):
    return lax.bitcast_convert_type(lax.shift_left(v, np.uint32(16)),
                                    jnp.float32)


def _unpack_hi(v):
    return lax.bitcast_convert_type(v & _HI_MASK, jnp.float32)


def _gated_message(g_, x):
    z = jnp.exp(-jnp.abs(x))
    eg = jnp.exp(-g_)
    q = 2.0 + z
    q2 = q * q
    q4 = q2 * q2
    q5 = q4 * q
    z2 = z * z
    z4 = z2 * z2
    num = (jnp.maximum(x, 0.0) * q5
           + 2.0 * z * (q4 + z2 * q2 * (1.0 / 3.0) + z4 * 0.2))
    return num / ((1.0 + eg) * q5)


def _sc_body(t_hbm, u_hbm, dd_hbm,
             src_hbm, dst_hbm, zeros_hbm, out_hbm,
             sidx, didx, trows, urows, ddv, mv, stage, acc,
             sem_t, sem_u, sem_i, sem_l):
    c = lax.axis_index("c")
    s = lax.axis_index("s")
    w = c * NS + s
    my_rows = s * ROWS_PER_TILE
    wbase = w * EPW

    pltpu.sync_copy(zeros_hbm, stage)
    for k in range(ROWS_PER_TILE // STAGE_ROWS):
        pltpu.sync_copy(
            stage, acc.at[pl.ds(my_rows + k * STAGE_ROWS, STAGE_ROWS)])
    plsc.subcore_barrier()

    def cbase(g):
        return pl.multiple_of(wbase + g * CHUNK, CHUNK)

    def issue_idx(g, b):
        base = cbase(g)
        pltpu.async_copy(src_hbm.at[pl.ds(base, CHUNK)], sidx[b], sem_i.at[b])
        pltpu.async_copy(dst_hbm.at[pl.ds(base, CHUNK)], didx[b], sem_i.at[b])

    def wait_idx(g, b):
        base = cbase(g)
        pltpu.make_async_copy(src_hbm.at[pl.ds(base, CHUNK)], sidx[b],
                              sem_i.at[b]).wait()
        pltpu.make_async_copy(dst_hbm.at[pl.ds(base, CHUNK)], didx[b],
                              sem_i.at[b]).wait()

    def issue_main(g, b):
        base = cbase(g)
        pltpu.async_copy(t_hbm.at[sidx[b]], trows[b], sem_t.at[b])
        pltpu.async_copy(u_hbm.at[didx[b]], urows[b], sem_u.at[b])
        pltpu.async_copy(dd_hbm.at[pl.ds(base, CHUNK)], ddv[b], sem_l.at[b])

    def wait_main(g, b):
        base = cbase(g)
        pltpu.make_async_copy(t_hbm.at[sidx[b]], trows[b], sem_t.at[b]).wait()
        pltpu.make_async_copy(u_hbm.at[didx[b]], urows[b], sem_u.at[b]).wait()
        pltpu.make_async_copy(dd_hbm.at[pl.ds(base, CHUNK)], ddv[b],
                              sem_l.at[b]).wait()

    issue_idx(0, 0)
    issue_idx(1, 1)
    wait_idx(0, 0)
    issue_main(0, 0)

    @pl.loop(0, NCHUNK, step=2)
    def chunk_body(i):
        for b in range(2):
            g = i + b
            wait_main(g, b)

            @pl.when(g + 1 < NCHUNK)
            def _():
                wait_idx(g + 1, 1 - b)
                issue_main(g + 1, 1 - b)

            @plsc.parallel_loop(0, CHUNK, unroll=4)
            def edge_body(e):
                for v in range(D // LANES):
                    lo = v * LANES
                    tp = trows[b][e, pl.ds(lo, LANES)]
                    up = urows[b][e, pl.ds(lo, LANES)]
                    dp = ddv[b][e, pl.ds(lo, LANES)]
                    g_ = _unpack_lo(tp) + _unpack_lo(up) + _unpack_lo(dp)
                    x = _unpack_hi(tp) + _unpack_hi(up) + _unpack_hi(dp)
                    mv[e, pl.ds(lo, LANES)] = _gated_message(g_, x)

            pltpu.sync_copy(mv, acc.at[didx[b]], add=True)

            @pl.when(g + 2 < NCHUNK)
            def _():
                issue_idx(g + 2, b)

    plsc.subcore_barrier()

    for k in range(ROWS_PER_TILE // STAGE_ROWS):
        rs = my_rows + k * STAGE_ROWS
        pltpu.sync_copy(acc.at[pl.ds(rs, STAGE_ROWS)], stage)
        obase = pl.multiple_of(c * NPAD + rs, STAGE_ROWS)
        pltpu.sync_copy(stage, out_hbm.at[pl.ds(obase, STAGE_ROWS)])


_sc_edges = functools.partial(
    pl.kernel,
    out_type=jax.ShapeDtypeStruct((NC * NPAD, D), jnp.float32),
    mesh=plsc.VectorSubcoreMesh(core_axis_name="c", subcore_axis_name="s"),
    scratch_types=[
        [pltpu.VMEM((CHUNK,), jnp.int32)] * 2,
        [pltpu.VMEM((CHUNK,), jnp.int32)] * 2,
        [pltpu.VMEM((CHUNK, D), jnp.uint32)] * 2,
        [pltpu.VMEM((CHUNK, D), jnp.uint32)] * 2,
        [pltpu.VMEM((CHUNK, D), jnp.uint32)] * 2,
        pltpu.VMEM((CHUNK, D), jnp.float32),
        pltpu.VMEM((STAGE_ROWS, D), jnp.float32),
        pltpu.VMEM_SHARED((NPAD, D), jnp.float32),
        pltpu.SemaphoreType.DMA((2,)),
        pltpu.SemaphoreType.DMA((2,)),
        pltpu.SemaphoreType.DMA((2,)),
        pltpu.SemaphoreType.DMA((2,)),
    ],
)(_sc_body)


def _final_body(f_ref, p_ref, o_ref):
    o_ref[...] = f_ref[...] + p_ref[0] + p_ref[1]


def _final_add(feature, partials):
    rb = 1000
    grid = N // rb
    partials = partials.reshape(NC, NPAD, D)
    return pl.pallas_call(
        _final_body,
        grid=(grid,),
        in_specs=[
            pl.BlockSpec((rb, D), lambda i: (i, 0)),
            pl.BlockSpec((NC, rb, D), lambda i: (0, i, 0)),
        ],
        out_specs=pl.BlockSpec((rb, D), lambda i: (i, 0)),
        out_shape=jax.ShapeDtypeStruct((N, D), jnp.float32),
    )(feature, partials)


def kernel(feature, edge_index, edge_dist, Wf, bf, Ws, bs):
    f32 = jnp.float32
    feature = feature.astype(f32)
    npad = E_PAD - E
    src = jnp.concatenate(
        [edge_index[0].astype(jnp.int32), jnp.zeros((npad,), jnp.int32)])
    dst = jnp.concatenate(
        [edge_index[1].astype(jnp.int32),
         jnp.full((npad,), PAD_DST, jnp.int32)])
    edge_dist_p = jnp.concatenate(
        [edge_dist.astype(f32), jnp.zeros((npad, ED), f32)])

    wgs = Wf[:, :D].T.astype(f32)
    wcs = Ws[:, :D].T.astype(f32)
    wgd = Wf[:, D:TWO_D].T.astype(f32)
    wcd = Ws[:, D:TWO_D].T.astype(f32)
    wge = Wf[:, TWO_D:].T.astype(f32)
    wce = Ws[:, TWO_D:].T.astype(f32)
    bg = bf.reshape(1, D).astype(f32)
    bc = bs.reshape(1, D).astype(f32)

    t_tab, u_tab = _node_tables(feature, wgs, wcs, wgd, wcd)
    dd = _edge_proj(edge_dist_p, wge, wce, bg, bc)
    zeros = jnp.zeros((STAGE_ROWS, D), f32)
    partials = _sc_edges(t_tab, u_tab, dd, src, dst, zeros)
    return _final_add(feature, partials)


# speedup vs baseline: 2.4257x; 1.3262x over previous
import functools

import jax
import jax.numpy as jnp
from jax import lax
from jax.experimental import pallas as pl
from jax.experimental.pallas import tpu as pltpu
from jax.experimental.pallas import tpu_sc as plsc

N = 10000
E = 320000
D = 128
HD = D // 2
ED = 10

NC = 2
NS = 16
LANES = 16
NW = NC * NS
EPW = 10240
E_PAD = NW * EPW
PAD_DST = 10200
CHUNK = 80
NCHUNK = EPW // CHUNK
NPAD = 10240
NACC = NPAD // 2
ROWS_PER_TILE = NACC // NS
STAGE_ROWS = 32


def _tables_body(f_ref, w0_ref, w1_ref, w2_ref, w3_ref,
                 t0_ref, t1_ref, u0_ref, u1_ref):
    x = f_ref[...]
    t0_ref[...] = jnp.dot(x, w0_ref[...], preferred_element_type=jnp.float32)
    t1_ref[...] = jnp.dot(x, w1_ref[...], preferred_element_type=jnp.float32)
    u0_ref[...] = jnp.dot(x, w2_ref[...], preferred_element_type=jnp.float32)
    u1_ref[...] = jnp.dot(x, w3_ref[...], preferred_element_type=jnp.float32)


def _node_tables(feature, w0, w1, w2, w3):
    rb = 1000
    grid = N // rb
    wspec = pl.BlockSpec((D, D), lambda i: (0, 0))
    ospec = pl.BlockSpec((rb, D), lambda i: (i, 0))
    oshape = jax.ShapeDtypeStruct((N, D), jnp.float32)
    return pl.pallas_call(
        _tables_body,
        grid=(grid,),
        in_specs=[pl.BlockSpec((rb, D), lambda i: (i, 0)),
                  wspec, wspec, wspec, wspec],
        out_specs=[ospec, ospec, ospec, ospec],
        out_shape=[oshape, oshape, oshape, oshape],
    )(feature, w0, w1, w2, w3)


def _edge_proj_body(ed_ref, wd0_ref, wd1_ref, b0_ref, b1_ref,
                    dd0_ref, dd1_ref):
    x = ed_ref[...]
    dd0_ref[...] = (
        jnp.dot(x, wd0_ref[...], preferred_element_type=jnp.float32)
        + b0_ref[...])
    dd1_ref[...] = (
        jnp.dot(x, wd1_ref[...], preferred_element_type=jnp.float32)
        + b1_ref[...])


def _edge_proj(edge_dist, wd0, wd1, b0, b1):
    eb = 4096
    grid = E_PAD // eb
    wspec = pl.BlockSpec((ED, D), lambda i: (0, 0))
    bspec = pl.BlockSpec((1, D), lambda i: (0, 0))
    ospec = pl.BlockSpec((eb, D), lambda i: (i, 0))
    oshape = jax.ShapeDtypeStruct((E_PAD, D), jnp.float32)
    return pl.pallas_call(
        _edge_proj_body,
        grid=(grid,),
        in_specs=[pl.BlockSpec((eb, ED), lambda i: (i, 0)),
                  wspec, wspec, bspec, bspec],
        out_specs=[ospec, ospec],
        out_shape=[oshape, oshape],
    )(edge_dist, wd0, wd1, b0, b1)


def _gated_message(g_, x):
    z = jnp.exp(-jnp.abs(x))
    eg = jnp.exp(-g_)
    q = 2.0 + z
    q2 = q * q
    q4 = q2 * q2
    q5 = q4 * q
    z2 = z * z
    z4 = z2 * z2
    num = (jnp.maximum(x, 0.0) * q5
           + 2.0 * z * (q4 + z2 * q2 * (1.0 / 3.0) + z4 * 0.2))
    return num / ((1.0 + eg) * q5)


def _sc_body(t0_hbm, t1_hbm, u0_hbm, u1_hbm, dd0_hbm, dd1_hbm,
             src_hbm, dst_hbm, d2_hbm, hm_hbm, zeros_hbm, out_hbm,
             sidx, didx, d2idx, hmv, trows, urows, ddv, mv, d2sc, stage, acc,
             sem_t, sem_u, sem_i, sem_l, sem_s):
    c = lax.axis_index("c")
    s = lax.axis_index("s")
    w = c * NS + s
    my_rows = s * ROWS_PER_TILE
    wbase = w * EPW

    for p, (t_hbm, u_hbm, dd_hbm) in enumerate(
            ((t0_hbm, u0_hbm, dd0_hbm), (t1_hbm, u1_hbm, dd1_hbm))):
        pltpu.sync_copy(zeros_hbm, stage)
        for k in range(ROWS_PER_TILE // STAGE_ROWS):
            pltpu.sync_copy(
                stage, acc.at[pl.ds(my_rows + k * STAGE_ROWS, STAGE_ROWS)])
        plsc.subcore_barrier()

        def cbase(g):
            return pl.multiple_of(wbase + g * CHUNK, CHUNK)

        def issue_idx(g, b):
            base = cbase(g)
            pltpu.async_copy(src_hbm.at[pl.ds(base, CHUNK)], sidx[b],
                             sem_i.at[b])
            pltpu.async_copy(dst_hbm.at[pl.ds(base, CHUNK)], didx[b],
                             sem_i.at[b])
            pltpu.async_copy(d2_hbm.at[pl.ds(base, CHUNK)], d2idx[b],
                             sem_i.at[b])

        def wait_idx(g, b):
            base = cbase(g)
            pltpu.make_async_copy(src_hbm.at[pl.ds(base, CHUNK)], sidx[b],
                                  sem_i.at[b]).wait()
            pltpu.make_async_copy(dst_hbm.at[pl.ds(base, CHUNK)], didx[b],
                                  sem_i.at[b]).wait()
            pltpu.make_async_copy(d2_hbm.at[pl.ds(base, CHUNK)], d2idx[b],
                                  sem_i.at[b]).wait()

        def issue_main(g, b):
            base = cbase(g)
            hbase = pl.multiple_of(base * LANES, CHUNK * LANES)
            pltpu.async_copy(t_hbm.at[sidx[b]], trows[b], sem_t.at[b])
            pltpu.async_copy(u_hbm.at[didx[b]], urows[b], sem_u.at[b])
            pltpu.async_copy(dd_hbm.at[pl.ds(base, CHUNK)], ddv[b],
                             sem_l.at[b])
            pltpu.async_copy(hm_hbm.at[pl.ds(hbase, CHUNK * LANES)], hmv[b],
                             sem_l.at[b])

        def wait_main(g, b):
            base = cbase(g)
            hbase = pl.multiple_of(base * LANES, CHUNK * LANES)
            pltpu.make_async_copy(t_hbm.at[sidx[b]], trows[b],
                                  sem_t.at[b]).wait()
            pltpu.make_async_copy(u_hbm.at[didx[b]], urows[b],
                                  sem_u.at[b]).wait()
            pltpu.make_async_copy(dd_hbm.at[pl.ds(base, CHUNK)], ddv[b],
                                  sem_l.at[b]).wait()
            pltpu.make_async_copy(hm_hbm.at[pl.ds(hbase, CHUNK * LANES)],
                                  hmv[b], sem_l.at[b]).wait()

        issue_idx(0, 0)
        issue_idx(1, 1)
        wait_idx(0, 0)
        issue_main(0, 0)

        def wait_scatter(b):
            pltpu.make_async_copy(mv[b], acc.at[d2sc[b]], sem_s.at[b]).wait()

        @pl.loop(0, NCHUNK, step=2)
        def chunk_body(i):
            for b in range(2):
                g = i + b
                wait_main(g, b)

                @pl.when(g + 1 < NCHUNK)
                def _():
                    wait_idx(g + 1, 1 - b)
                    issue_main(g + 1, 1 - b)

                @pl.when(g >= 2)
                def _():
                    wait_scatter(b)

                @plsc.parallel_loop(0, CHUNK, unroll=4)
                def edge_body(e):
                    hf = hmv[b][pl.ds(e * LANES, LANES)]
                    cf = 1.0 - hf
                    for v in range(HD // LANES):
                        lo = v * LANES
                        g_ = (trows[b][e, pl.ds(lo, LANES)]
                              + urows[b][e, pl.ds(lo, LANES)]
                              + ddv[b][e, pl.ds(lo, LANES)])
                        x = (trows[b][e, pl.ds(HD + lo, LANES)]
                             + urows[b][e, pl.ds(HD + lo, LANES)]
                             + ddv[b][e, pl.ds(HD + lo, LANES)])
                        m = _gated_message(g_, x)
                        mv[b][e, pl.ds(lo, LANES)] = m * cf
                        mv[b][e, pl.ds(HD + lo, LANES)] = m * hf

                for j in range(CHUNK // LANES):
                    jl = j * LANES
                    d2sc[b][pl.ds(jl, LANES)] = d2idx[b][pl.ds(jl, LANES)]
                pltpu.async_copy(mv[b], acc.at[d2sc[b]], sem_s.at[b],
                                 add=True)

                @pl.when(g + 2 < NCHUNK)
                def _():
                    issue_idx(g + 2, b)

        wait_scatter(0)
        wait_scatter(1)
        plsc.subcore_barrier()

        for k in range(ROWS_PER_TILE // STAGE_ROWS):
            rs = my_rows + k * STAGE_ROWS
            pltpu.sync_copy(acc.at[pl.ds(rs, STAGE_ROWS)], stage)
            obase = pl.multiple_of((c * 2 + p) * NACC + rs, STAGE_ROWS)
            pltpu.sync_copy(stage, out_hbm.at[pl.ds(obase, STAGE_ROWS)])
        plsc.subcore_barrier()


_sc_edges = functools.partial(
    pl.kernel,
    out_type=jax.ShapeDtypeStruct((NC * 2 * NACC, D), jnp.float32),
    mesh=plsc.VectorSubcoreMesh(core_axis_name="c", subcore_axis_name="s"),
    scratch_types=[
        [pltpu.VMEM((CHUNK,), jnp.int32)] * 2,
        [pltpu.VMEM((CHUNK,), jnp.int32)] * 2,
        [pltpu.VMEM((CHUNK,), jnp.int32)] * 2,
        [pltpu.VMEM((CHUNK * LANES,), jnp.float32)] * 2,
        [pltpu.VMEM((CHUNK, D), jnp.float32)] * 2,
        [pltpu.VMEM((CHUNK, D), jnp.float32)] * 2,
        [pltpu.VMEM((CHUNK, D), jnp.float32)] * 2,
        [pltpu.VMEM((CHUNK, D), jnp.float32)] * 2,
        [pltpu.VMEM((CHUNK,), jnp.int32)] * 2,
        pltpu.VMEM((STAGE_ROWS, D), jnp.float32),
        pltpu.VMEM_SHARED((NACC, D), jnp.float32),
        pltpu.SemaphoreType.DMA((2,)),
        pltpu.SemaphoreType.DMA((2,)),
        pltpu.SemaphoreType.DMA((2,)),
        pltpu.SemaphoreType.DMA((2,)),
        pltpu.SemaphoreType.DMA((2,)),
    ],
)(_sc_body)


def _final_body(f_ref, p_ref, o_ref):
    lo = p_ref[0, 0] + p_ref[1, 0]
    hi = p_ref[0, 1] + p_ref[1, 1]
    o_ref[...] = f_ref[...] + jnp.concatenate([lo, hi], axis=-1)


def _final_add(feature, partials):
    rb = 1000
    grid = N // rb
    partials = partials.reshape(NC, 2, NPAD, HD)
    return pl.pallas_call(
        _final_body,
        grid=(grid,),
        in_specs=[
            pl.BlockSpec((rb, D), lambda i: (i, 0)),
            pl.BlockSpec((NC, 2, rb, HD), lambda i: (0, 0, i, 0)),
        ],
        out_specs=pl.BlockSpec((rb, D), lambda i: (i, 0)),
        out_shape=jax.ShapeDtypeStruct((N, D), jnp.float32),
    )(feature, partials)


def kernel(feature, edge_index, edge_dist, Wf, bf, Ws, bs):
    f32 = jnp.float32
    feature = feature.astype(f32)
    npad = E_PAD - E
    src = jnp.concatenate(
        [edge_index[0].astype(jnp.int32), jnp.zeros((npad,), jnp.int32)])
    dst = jnp.concatenate(
        [edge_index[1].astype(jnp.int32),
         jnp.full((npad,), PAD_DST, jnp.int32)])
    d2 = lax.shift_right_logical(dst, 1)
    hmask = jnp.repeat((dst & 1).astype(f32), LANES)
    edge_dist_p = jnp.concatenate(
        [edge_dist.astype(f32), jnp.zeros((npad, ED), f32)])

    def wpair(lo):
        return jnp.concatenate([Wf[lo:lo + HD], Ws[lo:lo + HD]],
                               axis=0).astype(f32)

    w_t0 = wpair(0)[:, :D].T
    w_t1 = wpair(HD)[:, :D].T
    w_u0 = wpair(0)[:, D:2 * D].T
    w_u1 = wpair(HD)[:, D:2 * D].T
    wd0 = wpair(0)[:, 2 * D:].T
    wd1 = wpair(HD)[:, 2 * D:].T
    b0 = jnp.concatenate([bf[:HD], bs[:HD]]).reshape(1, D).astype(f32)
    b1 = jnp.concatenate([bf[HD:], bs[HD:]]).reshape(1, D).astype(f32)

    t0, t1, u0, u1 = _node_tables(feature, w_t0, w_t1, w_u0, w_u1)
    dd0, dd1 = _edge_proj(edge_dist_p, wd0, wd1, b0, b1)
    zeros = jnp.zeros((STAGE_ROWS, D), f32)
    partials = _sc_edges(t0, t1, u0, u1, dd0, dd1, src, dst, d2, hmask, zeros)
    return _final_add(feature, partials)
